# baseline (speedup 1.0000x reference)
import functools

import jax
import jax.numpy as jnp
from jax import lax
from jax.experimental import pallas as pl
from jax.experimental.pallas import tpu as pltpu
from jax.experimental.pallas import tpu_sc as plsc

_EPS = 1e-08
_L = 16
_TC = 25
_W = 512

_GATHER_DNUMS = lax.GatherDimensionNumbers(
    offset_dims=(), collapsed_slice_dims=(0,), start_index_map=(0,))


def _iota():
    return lax.iota(jnp.int32, _L)


def _shuf(v, idx):
    return lax.gather(v, lax.reshape(idx, (_L, 1)), _GATHER_DNUMS,
                      slice_sizes=(1,),
                      mode=lax.GatherScatterMode.PROMISE_IN_BOUNDS)


def _rot(v, r):
    return _shuf(v, (_iota() - r) & (_L - 1))


def _hsum_tree(vs):
    it = _iota()
    k = _L // 2
    while len(vs) > 1:
        nvs = []
        m = (it & k) == 0
        xk = it ^ k
        for i in range(0, len(vs), 2):
            a, b = vs[i], vs[i + 1]
            c = (jnp.where(m, a, _shuf(b, xk))
                 + jnp.where(m, _shuf(a, xk), b))
            nvs.append(c)
        vs = nvs
        k //= 2
    rev = (((it & 1) << 3) | ((it & 2) << 1)
           | ((it & 4) >> 1) | ((it & 8) >> 3))
    return _shuf(vs[0], rev)


def _rsqrt(a):
    i = lax.bitcast_convert_type(a, jnp.int32)
    i = jnp.int32(0x5F3759DF) - lax.shift_right_logical(i, 1)
    y = lax.bitcast_convert_type(i, jnp.float32)
    for _ in range(3):
        y = y * (1.5 - 0.5 * a * y * y)
    return y


@functools.partial(jax.jit, static_argnums=(1, 2))
def _sc_repack(tokT, V, D):
    info = plsc.get_sparse_core_info()
    NC, NS = info.num_cores, info.num_subcores
    NW = NC * NS
    per_w = V // NW
    nblk = (per_w + _L - 1 + _W - 1) // _W

    mesh = plsc.VectorSubcoreMesh(core_axis_name="c", subcore_axis_name="s")

    @functools.partial(
        pl.kernel,
        mesh=mesh,
        compiler_params=pltpu.CompilerParams(
            use_tc_tiling_on_sc=False, needs_layout_passes=False),
        out_type=jax.ShapeDtypeStruct((V, D), jnp.float32),
        scratch_types=[
            pltpu.VMEM((D, _W), jnp.float32),
            pltpu.VMEM((_W, D), jnp.float32),
            pltpu.VMEM((D, _L), jnp.float32),
        ],
    )
    def rk(tokT_h, tokR_h, blk_v, obuf_v, swz_v):
        wid = lax.axis_index("s") * NC + lax.axis_index("c")
        sA = (wid * per_w) // _L * _L

        def blk_body(i, carry):
            s = jnp.minimum(sA + i * _W, V - _W)
            pltpu.sync_copy(tokT_h.at[:, pl.ds(s, _W)], blk_v)

            def grp_body(u, carry2):
                it = _iota()
                for c in range(D):
                    v = blk_v[c, pl.ds(u * _L, _L)]
                    swz_v[c, :] = _rot(v, c)
                for i16 in range(_L):
                    for q in range(D // _L):
                        rows = q * _L + it
                        cols = (i16 + q * _L + it) & (_L - 1)
                        g = plsc.load_gather(swz_v, [rows, cols])
                        obuf_v[u * _L + i16, pl.ds(q * _L, _L)] = g
                return carry2

            lax.fori_loop(0, _W // _L, grp_body, 0)
            pltpu.sync_copy(obuf_v, tokR_h.at[pl.ds(s, _W)])
            return carry

        lax.fori_loop(0, nblk, blk_body, 0)

    return rk(tokT)


@functools.partial(jax.jit, static_argnums=(4, 5, 6))
def _sc_embed(xT, tokR, pos_table, scale, T, D, B):
    info = plsc.get_sparse_core_info()
    NC, NS = info.num_cores, info.num_subcores
    NW = NC * NS
    BPW = B // NW
    NG = BPW // _L
    NCH = T // _TC
    PMAX = 2 * ((T + _L - 1) // _L * _L)

    mesh = plsc.VectorSubcoreMesh(core_axis_name="c", subcore_axis_name="s")

    @functools.partial(
        pl.kernel,
        mesh=mesh,
        compiler_params=pltpu.CompilerParams(
            use_tc_tiling_on_sc=False, needs_layout_passes=False),
        out_type=jax.ShapeDtypeStruct((T, D, B), jnp.float32),
        scratch_types=[
            pltpu.VMEM((T, _L), jnp.int32),
            pltpu.VMEM((_TC * _L,), jnp.int32),
            pltpu.VMEM((_TC * _L, D), jnp.float32),
            pltpu.VMEM((_TC, D, _L), jnp.float32),
            pltpu.VMEM((_L, D), jnp.float32),
            pltpu.VMEM((PMAX, D), jnp.float32),
            pltpu.VMEM((D,), jnp.float32),
            pltpu.SemaphoreType.DMA,
        ],
    )
    def k(xT_h, tokR_h, pos_h, scale_h, out_h,
          idx_v, idxg_v, rows_v, st_v, swz_v, pos_v, scale_v, sem):
        wid = lax.axis_index("s") * NC + lax.axis_index("c")
        pltpu.sync_copy(pos_h.at[pl.ds(0, PMAX)], pos_v)
        pltpu.sync_copy(scale_h, scale_v)
        svals = [scale_v[pl.ds(j * _L, _L)][i]
                 for j in range(D // _L) for i in range(_L)]
        b0w = wid * BPW

        def group_body(g, carry):
            b0 = b0w + g * _L
            pltpu.sync_copy(xT_h.at[:, pl.ds(b0, _L)], idx_v)

            def chunk_body(c, carry2):
                t0 = c * _TC

                def prep_body(toff, carry3):
                    idxg_v[pl.ds(toff * _L, _L)] = idx_v[t0 + toff, :]
                    return carry3

                lax.fori_loop(0, _TC, prep_body, 0)
                pltpu.async_copy(tokR_h.at[idxg_v], rows_v, sem).wait()

                def comp_body(toff, carry3):
                    t = t0 + toff
                    it = _iota()
                    pos_j = [pos_v[t, pl.ds(j * _L, _L)]
                             for j in range(D // _L)]
                    accs = []
                    for r in range(_L):
                        row = toff * _L + r
                        acc_r = None
                        for j in range(D // _L):
                            v = rows_v[row, pl.ds(j * _L, _L)] + pos_j[j]
                            swz_v[r, pl.ds(j * _L, _L)] = _rot(v, r)
                            sq = v * v
                            acc_r = sq if acc_r is None else acc_r + sq
                        accs.append(acc_r)
                    acc = _hsum_tree(accs)
                    rn = _rsqrt(acc + D * _EPS) * float(D) ** 0.5
                    for d in range(D):
                        q, ci = d // _L, d % _L
                        cols = q * _L + ((ci + it) & (_L - 1))
                        g2 = plsc.load_gather(swz_v, [it, cols])
                        st_v[toff, d, :] = g2 * rn * svals[d]
                    return carry3

                lax.fori_loop(0, _TC, comp_body, 0)
                pltpu.sync_copy(
                    st_v, out_h.at[pl.ds(t0, _TC), :, pl.ds(b0, _L)])
                return carry2

            lax.fori_loop(0, NCH, chunk_body, 0)
            return carry

        lax.fori_loop(0, NG, group_body, 0)

    return k(xT, tokR, pos_table, scale)


def kernel(x, tok_table, pos_table, scale):
    Bz, Tz = x.shape
    V, D = tok_table.shape
    tokR = _sc_repack(tok_table.T, V, D)
    out3 = _sc_embed(x.T, tokR, pos_table, scale, Tz, D, Bz)
    return jnp.transpose(out3, (2, 0, 1))


# speedup vs baseline: 4.7965x; 4.7965x over previous
import functools

import jax
import jax.numpy as jnp
from jax import lax
from jax.experimental import pallas as pl
from jax.experimental.pallas import tpu as pltpu
from jax.experimental.pallas import tpu_sc as plsc

_EPS = 1e-08
_L = 16

_GATHER_DNUMS = lax.GatherDimensionNumbers(
    offset_dims=(), collapsed_slice_dims=(0,), start_index_map=(0,))


def _lane_sum(v):
    lanes = lax.iota(jnp.int32, _L)
    for k in (8, 4, 2, 1):
        perm = lax.reshape(jnp.bitwise_xor(lanes, k), (_L, 1))
        v = v + lax.gather(v, perm, _GATHER_DNUMS, slice_sizes=(1,),
                           mode=lax.GatherScatterMode.PROMISE_IN_BOUNDS)
    return v


def _rsqrt(a):
    i = lax.bitcast_convert_type(a, jnp.int32)
    i = jnp.int32(0x5F3759DF) - lax.shift_right_logical(i, 1)
    y = lax.bitcast_convert_type(i, jnp.float32)
    h = 0.5 * a
    for _ in range(2):
        y = y * (1.5 - h * y * y)
    return y


@functools.partial(jax.jit, static_argnums=(4, 5))
def _sc_embed(x_flat, tok_table, pos_table, scale, T, D):
    info = plsc.get_sparse_core_info()
    NC, NS = info.num_cores, info.num_subcores
    NW = NC * NS
    N = x_flat.shape[0]
    seq_per_w = N // T // NW
    nj = D // _L

    mesh = plsc.VectorSubcoreMesh(core_axis_name="c", subcore_axis_name="s")

    @functools.partial(
        pl.kernel,
        mesh=mesh,
        compiler_params=pltpu.CompilerParams(
            use_tc_tiling_on_sc=False, needs_layout_passes=False),
        out_type=jax.ShapeDtypeStruct((N // 2, 2 * D), jnp.float32),
        scratch_types=[
            pltpu.VMEM((T,), jnp.int32),
            pltpu.VMEM((T, D), jnp.float32),
            pltpu.VMEM((T // 2, 2 * D), jnp.float32),
            pltpu.VMEM((T, D), jnp.float32),
            pltpu.VMEM((D,), jnp.float32),
            pltpu.SemaphoreType.DMA,
        ],
    )
    def k(x_hbm, tok_hbm, pos_hbm, scale_hbm, out_hbm,
          idx_v, rows_v, out_v, pos_v, scale_v, sem):
        wid = lax.axis_index("s") * NC + lax.axis_index("c")
        pltpu.sync_copy(pos_hbm.at[pl.ds(0, T)], pos_v)
        pltpu.sync_copy(scale_hbm, scale_v)
        s8 = [scale_v[pl.ds(j * _L, _L)] * float(D) ** 0.5 for j in range(nj)]
        base_w = wid * seq_per_w * T

        def seq_body(c, carry):
            base = base_w + c * T
            pltpu.sync_copy(x_hbm.at[pl.ds(base, T)], idx_v)
            pltpu.async_copy(tok_hbm.at[idx_v], rows_v, sem).wait()

            def row_pair(tp, carry2):
                for half in range(2):
                    t = tp * 2 + half
                    vs = []
                    acc = None
                    for j in range(nj):
                        v = (rows_v[t, pl.ds(j * _L, _L)]
                             + pos_v[t, pl.ds(j * _L, _L)])
                        vs.append(v)
                        sq = v * v
                        acc = sq if acc is None else acc + sq
                    rn = _rsqrt(_lane_sum(acc) + D * _EPS)
                    for j in range(nj):
                        out_v[tp, pl.ds(half * D + j * _L, _L)] = (
                            vs[j] * (rn * s8[j]))
                return carry2

            lax.fori_loop(0, T // 2, row_pair, 0)
            pltpu.sync_copy(out_v, out_hbm.at[pl.ds(base // 2, T // 2)])
            return carry

        lax.fori_loop(0, seq_per_w, seq_body, 0)

    return k(x_flat, tok_table, pos_table, scale)


def kernel(x, tok_table, pos_table, scale):
    Bz, Tz = x.shape
    D = tok_table.shape[1]
    out2 = _sc_embed(x.reshape(Bz * Tz), tok_table, pos_table, scale, Tz, D)
    return out2.reshape(Bz, Tz, D)
